# baseline (speedup 1.0000x reference)
import jax
import jax.numpy as jnp
from jax import lax
from jax.experimental import pallas as pl
from jax.experimental.pallas import tpu as pltpu


def _tdo_body(lam_ref, wq_ref, wk_ref, qt_ref, qp_ref, kt_ref, kp_ref,
              v_ref, o_ref):
    hd = qt_ref.shape[2]
    nf = lam_ref.shape[1]
    qt = qt_ref[0]
    qp = qp_ref[0]
    kt = kt_ref[0]
    kp = kp_ref[0]
    v = v_ref[0]
    lam_row = lam_ref[...]

    def per_token_lambda(x, xp, w):
        scores = (jnp.dot(x, w[:hd], preferred_element_type=jnp.float32)
                  + jnp.dot(xp, w[hd:], preferred_element_type=jnp.float32))
        m = jnp.max(scores, axis=-1, keepdims=True)
        iota = lax.broadcasted_iota(jnp.int32, scores.shape, 1)
        idx = jnp.min(jnp.where(scores == m, iota, nf), axis=-1,
                      keepdims=True)
        return jnp.sum(jnp.where(iota == idx, lam_row, 0.0), axis=-1,
                       keepdims=True)

    lam_q = per_token_lambda(qt, qp, wq_ref[...])
    lam_k = per_token_lambda(kt, kp, wk_ref[...])
    q_diff = qt - lam_q * qp
    k_diff = kt - lam_k * kp
    kv = lax.dot_general(k_diff, v, (((0,), (0,)), ((), ())),
                         preferred_element_type=jnp.float32)
    o_ref[0] = jnp.dot(q_diff, kv, preferred_element_type=jnp.float32)


def kernel(Q_t, Q_prime_t, K_t, K_prime_t, V, lambdas, Wq, Wk):
    b, h, s, hd = Q_t.shape
    bh = b * h
    nf = lambdas.shape[0]
    dim = Wq.shape[1]

    qt = Q_t.reshape(bh, s, hd)
    qp = Q_prime_t.reshape(bh, s, hd)
    kt = K_t.reshape(bh, s, hd)
    kp = K_prime_t.reshape(bh, s, hd)
    v = V.reshape(bh, s, hd)
    lam = lambdas.reshape(1, nf)
    wq_t = Wq.T
    wk_t = Wk.T

    big = pl.BlockSpec((1, s, hd), lambda i: (i, 0, 0))
    lam_spec = pl.BlockSpec((1, nf), lambda i: (0, 0))
    w_spec = pl.BlockSpec((dim, nf), lambda i: (0, 0))

    out = pl.pallas_call(
        _tdo_body,
        out_shape=jax.ShapeDtypeStruct((bh, s, hd), jnp.float32),
        grid=(bh,),
        in_specs=[lam_spec, w_spec, w_spec, big, big, big, big, big],
        out_specs=big,
        compiler_params=pltpu.CompilerParams(
            dimension_semantics=("parallel",),
        ),
        name="token_diff_op",
    )(lam, wq_t, wk_t, qt, qp, kt, kp, v)
    return out.reshape(b, h, s, hd)


# speedup vs baseline: 1.0863x; 1.0863x over previous
import jax
import jax.numpy as jnp
from jax import lax
from jax.experimental import pallas as pl
from jax.experimental.pallas import tpu as pltpu


def _tdo_body(lam_mat_ref, wq_ref, wk_ref, qt_ref, qp_ref, kt_ref, kp_ref,
              v_ref, o_ref):
    hd = qt_ref.shape[2]
    nf = lam_mat_ref.shape[0]
    qt = qt_ref[0]
    qp = qp_ref[0]
    kt = kt_ref[0]
    kp = kp_ref[0]
    v = v_ref[0]
    lam_mat = lam_mat_ref[...]

    def lambda_bcast(x, xp, w):
        scores = (jnp.dot(x, w[:hd], preferred_element_type=jnp.float32)
                  + jnp.dot(xp, w[hd:], preferred_element_type=jnp.float32))
        m = jnp.max(scores, axis=-1, keepdims=True)
        iota = lax.broadcasted_iota(jnp.int32, scores.shape, 1
                                    ).astype(jnp.float32)
        idx = jnp.min(jnp.where(scores == m, iota, float(nf)), axis=-1,
                      keepdims=True)
        one_hot = jnp.where(iota == idx, 1.0, 0.0)
        return jnp.dot(one_hot, lam_mat,
                       preferred_element_type=jnp.float32)

    q_diff = qt - lambda_bcast(qt, qp, wq_ref[...]) * qp
    k_diff = kt - lambda_bcast(kt, kp, wk_ref[...]) * kp
    kv = lax.dot_general(k_diff, v, (((0,), (0,)), ((), ())),
                         preferred_element_type=jnp.float32)
    o_ref[0] = jnp.dot(q_diff, kv, preferred_element_type=jnp.float32)


def kernel(Q_t, Q_prime_t, K_t, K_prime_t, V, lambdas, Wq, Wk):
    b, h, s, hd = Q_t.shape
    bh = b * h
    nf = lambdas.shape[0]
    dim = Wq.shape[1]

    qt = Q_t.reshape(bh, s, hd)
    qp = Q_prime_t.reshape(bh, s, hd)
    kt = K_t.reshape(bh, s, hd)
    kp = K_prime_t.reshape(bh, s, hd)
    v = V.reshape(bh, s, hd)
    lam_mat = jnp.broadcast_to(lambdas[:, None], (nf, hd))
    wq_t = Wq.T
    wk_t = Wk.T

    big = pl.BlockSpec((1, s, hd), lambda i: (i, 0, 0))
    lam_spec = pl.BlockSpec((nf, hd), lambda i: (0, 0))
    w_spec = pl.BlockSpec((dim, nf), lambda i: (0, 0))

    out = pl.pallas_call(
        _tdo_body,
        out_shape=jax.ShapeDtypeStruct((bh, s, hd), jnp.float32),
        grid=(bh,),
        in_specs=[lam_spec, w_spec, w_spec, big, big, big, big, big],
        out_specs=big,
        compiler_params=pltpu.CompilerParams(
            dimension_semantics=("parallel",),
        ),
        name="token_diff_op",
    )(lam_mat, wq_t, wk_t, qt, qp, kt, kp, v)
    return out.reshape(b, h, s, hd)


# speedup vs baseline: 4.9496x; 4.5564x over previous
import jax
import jax.numpy as jnp
from jax import lax
from jax.experimental import pallas as pl
from jax.experimental.pallas import tpu as pltpu


def _tdo_body(lam_ref, wq_ref, wk_ref, qt_ref, qp_ref, kt_ref, kp_ref,
              v_ref, o_ref):
    hd = qt_ref.shape[-2]
    nf = wq_ref.shape[0]
    qt = qt_ref[0, 0]
    qp = qp_ref[0, 0]
    kt = kt_ref[0, 0]
    kp = kp_ref[0, 0]
    v = v_ref[0, 0]

    def lambda_row(x, xp, w):
        scores = (
            lax.dot_general(w[:, :hd], x, (((1,), (0,)), ((), ())),
                            preferred_element_type=jnp.float32)
            + lax.dot_general(w[:, hd:], xp, (((1,), (0,)), ((), ())),
                              preferred_element_type=jnp.float32))
        m = jnp.max(scores, axis=0, keepdims=True)
        iota = lax.broadcasted_iota(jnp.int32, scores.shape, 0)
        idx = jnp.min(jnp.where(scores == m, iota, nf), axis=0,
                      keepdims=True)
        lam = jnp.zeros((1, idx.shape[1]), jnp.float32)
        for n in range(nf):
            lam = jnp.where(idx == n, lam_ref[0, n], lam)
        return lam

    q_diff = qt - lambda_row(qt, qp, wq_ref[...]) * qp
    k_diff = kt - lambda_row(kt, kp, wk_ref[...]) * kp
    kvt = lax.dot_general(v, k_diff, (((1,), (1,)), ((), ())),
                          preferred_element_type=jnp.float32)
    o_ref[0, 0] = jnp.dot(kvt, q_diff, preferred_element_type=jnp.float32)


def kernel(Q_t, Q_prime_t, K_t, K_prime_t, V, lambdas, Wq, Wk):
    b, h, s, hd = Q_t.shape
    nf = lambdas.shape[0]
    dim = Wq.shape[1]

    qt = jnp.swapaxes(Q_t, 2, 3)
    qp = jnp.swapaxes(Q_prime_t, 2, 3)
    kt = jnp.swapaxes(K_t, 2, 3)
    kp = jnp.swapaxes(K_prime_t, 2, 3)
    v = jnp.swapaxes(V, 2, 3)
    lam = lambdas.reshape(1, nf)

    big = pl.BlockSpec((1, 1, hd, s), lambda i: (i // h, i % h, 0, 0))
    lam_spec = pl.BlockSpec(memory_space=pltpu.SMEM)
    w_spec = pl.BlockSpec((nf, dim), lambda i: (0, 0))

    out = pl.pallas_call(
        _tdo_body,
        out_shape=jax.ShapeDtypeStruct((b, h, hd, s), jnp.float32),
        grid=(b * h,),
        in_specs=[lam_spec, w_spec, w_spec, big, big, big, big, big],
        out_specs=big,
        compiler_params=pltpu.CompilerParams(
            dimension_semantics=("parallel",),
        ),
        name="token_diff_op",
    )(lam, Wq, Wk, qt, qp, kt, kp, v)
    return jnp.swapaxes(out, 2, 3)


# speedup vs baseline: 5.5909x; 1.1296x over previous
import jax
import jax.numpy as jnp
from jax import lax
from jax.experimental import pallas as pl
from jax.experimental.pallas import tpu as pltpu


def _tdo_body(lam_ref, wq_ref, wk_ref, qt_ref, qp_ref, kt_ref, kp_ref,
              v_ref, o_ref):
    hd = qt_ref.shape[-2]
    nf = wq_ref.shape[0]

    def lambda_row(x, xp, w):
        scores = (
            lax.dot_general(w[:, :hd], x, (((1,), (0,)), ((), ())),
                            preferred_element_type=jnp.float32)
            + lax.dot_general(w[:, hd:], xp, (((1,), (0,)), ((), ())),
                              preferred_element_type=jnp.float32))
        m = jnp.max(scores, axis=0, keepdims=True)
        iota = lax.broadcasted_iota(jnp.int32, scores.shape, 0)
        idx = jnp.min(jnp.where(scores == m, iota, nf), axis=0,
                      keepdims=True)
        lam = jnp.zeros((1, idx.shape[1]), jnp.float32)
        for n in range(nf):
            lam = jnp.where(idx == n, lam_ref[0, n], lam)
        return lam

    for j in range(qt_ref.shape[1]):
        qt = qt_ref[0, j]
        qp = qp_ref[0, j]
        kt = kt_ref[0, j]
        kp = kp_ref[0, j]
        v = v_ref[0, j]
        q_diff = qt - lambda_row(qt, qp, wq_ref[...]) * qp
        k_diff = kt - lambda_row(kt, kp, wk_ref[...]) * kp
        kvt = lax.dot_general(v, k_diff, (((1,), (1,)), ((), ())),
                              preferred_element_type=jnp.float32)
        o_ref[0, j] = jnp.dot(kvt, q_diff,
                              preferred_element_type=jnp.float32)


def kernel(Q_t, Q_prime_t, K_t, K_prime_t, V, lambdas, Wq, Wk):
    b, h, s, hd = Q_t.shape
    nf = lambdas.shape[0]
    dim = Wq.shape[1]

    qt = jnp.swapaxes(Q_t, 2, 3)
    qp = jnp.swapaxes(Q_prime_t, 2, 3)
    kt = jnp.swapaxes(K_t, 2, 3)
    kp = jnp.swapaxes(K_prime_t, 2, 3)
    v = jnp.swapaxes(V, 2, 3)
    lam = lambdas.reshape(1, nf)

    hb = 2
    hg = h // hb
    big = pl.BlockSpec((1, hb, hd, s), lambda i: (i // hg, i % hg, 0, 0))
    lam_spec = pl.BlockSpec(memory_space=pltpu.SMEM)
    w_spec = pl.BlockSpec((nf, dim), lambda i: (0, 0))

    out = pl.pallas_call(
        _tdo_body,
        out_shape=jax.ShapeDtypeStruct((b, h, hd, s), jnp.float32),
        grid=(b * hg,),
        in_specs=[lam_spec, w_spec, w_spec, big, big, big, big, big],
        out_specs=big,
        compiler_params=pltpu.CompilerParams(
            dimension_semantics=("parallel",),
        ),
        name="token_diff_op",
    )(lam, Wq, Wk, qt, qp, kt, kp, v)
    return jnp.swapaxes(out, 2, 3)
